# baseline (speedup 1.0000x reference)
import functools

import jax
import jax.numpy as jnp
from jax import lax
from jax.experimental import pallas as pl
from jax.experimental.pallas import tpu as pltpu
from jax.experimental.pallas import tpu_sc as plsc

_CHUNK = 128


@functools.lru_cache(maxsize=None)
def _make_gather(B, D):
    info = plsc.get_sparse_core_info()
    NC, NS = info.num_cores, info.num_subcores
    NW = NC * NS
    n_chunks_total = B // _CHUNK
    assert n_chunks_total % NW == 0
    n_per_w = n_chunks_total // NW
    mesh = plsc.VectorSubcoreMesh(core_axis_name="c", subcore_axis_name="s")

    @functools.partial(
        pl.kernel,
        mesh=mesh,
        compiler_params=pltpu.CompilerParams(use_tc_tiling_on_sc=False),
        out_type=jax.ShapeDtypeStruct((B, D), jnp.float32),
        scratch_types=[
            pltpu.VMEM((n_per_w, _CHUNK), jnp.int32),
            pltpu.VMEM((_CHUNK, D), jnp.float32),
            pltpu.SemaphoreType.DMA,
        ],
    )
    def body(idx_hbm, table_hbm, out_hbm, idx_v, rows_v, sem):
        wid = lax.axis_index("s") * NC + lax.axis_index("c")
        row0 = wid * n_per_w
        pltpu.sync_copy(idx_hbm.at[pl.ds(row0, n_per_w)], idx_v)

        def step(j, carry):
            pltpu.async_copy(table_hbm.at[idx_v.at[j]], rows_v, sem).wait()
            pltpu.sync_copy(rows_v, out_hbm.at[pl.ds((row0 + j) * _CHUNK, _CHUNK)])
            return carry

        lax.fori_loop(0, n_per_w, step, 0)

    return body


def kernel(input_ids, weight):
    B = input_ids.size
    D = weight.shape[1]
    idx = input_ids.reshape(B // _CHUNK, _CHUNK).astype(jnp.int32)
    out = _make_gather(B, D)(idx, weight)
    return out.reshape(*input_ids.shape, D)


# speedup vs baseline: 1.1166x; 1.1166x over previous
import functools

import jax
import jax.numpy as jnp
from jax import lax
from jax.experimental import pallas as pl
from jax.experimental.pallas import tpu as pltpu
from jax.experimental.pallas import tpu_sc as plsc

_CHUNK = 128


@functools.lru_cache(maxsize=None)
def _make_gather(B, D):
    info = plsc.get_sparse_core_info()
    NC, NS = info.num_cores, info.num_subcores
    NW = NC * NS
    n_chunks_total = B // _CHUNK
    assert n_chunks_total % NW == 0
    n_per_w = n_chunks_total // NW
    mesh = plsc.VectorSubcoreMesh(core_axis_name="c", subcore_axis_name="s")

    K = 4
    assert n_per_w % (2 * K) == 0
    npair = n_per_w // (2 * K)

    @functools.partial(
        pl.kernel,
        mesh=mesh,
        compiler_params=pltpu.CompilerParams(use_tc_tiling_on_sc=False),
        out_type=jax.ShapeDtypeStruct((B, D), jnp.float32),
        scratch_types=[
            pltpu.VMEM((n_per_w, _CHUNK), jnp.int32),
            pltpu.VMEM((2, K, _CHUNK, D), jnp.float32),
            pltpu.SemaphoreType.DMA,
            pltpu.SemaphoreType.DMA,
            pltpu.SemaphoreType.DMA,
        ],
    )
    def body(idx_hbm, table_hbm, out_hbm, idx_v, rows_v, gsem_a, gsem_b, wsem):
        wid = lax.axis_index("s") * NC + lax.axis_index("c")
        row0 = wid * n_per_w
        pltpu.sync_copy(idx_hbm.at[pl.ds(row0, n_per_w)], idx_v)

        gsems = (gsem_a, gsem_b)

        def fire_gathers(g, half):
            for b in range(K):
                pltpu.async_copy(
                    table_hbm.at[idx_v.at[g * K + b]],
                    rows_v.at[half].at[b],
                    gsems[half],
                )

        def drain_gathers(half):
            for b in range(K):
                pltpu.make_async_copy(
                    out_hbm.at[pl.ds(0, _CHUNK)], rows_v.at[half].at[b], gsems[half]
                ).wait()

        def fire_writebacks(g, half):
            for b in range(K):
                c = g * K + b
                pltpu.async_copy(
                    rows_v.at[half].at[b],
                    out_hbm.at[pl.ds((row0 + c) * _CHUNK, _CHUNK)],
                    wsem,
                )

        def drain_writebacks(half):
            for b in range(K):
                pltpu.make_async_copy(
                    rows_v.at[half].at[b], out_hbm.at[pl.ds(0, _CHUNK)], wsem
                ).wait()

        fire_gathers(0, 0)

        def pair(g2, carry):
            g = g2 * 2
            fire_gathers(g + 1, 1)
            drain_gathers(0)
            fire_writebacks(g, 0)
            drain_writebacks(0)
            fire_gathers(g + 2, 0)
            drain_gathers(1)
            fire_writebacks(g + 1, 1)
            drain_writebacks(1)
            return carry

        lax.fori_loop(0, npair - 1, pair, 0)

        g = (npair - 1) * 2
        fire_gathers(g + 1, 1)
        drain_gathers(0)
        fire_writebacks(g, 0)
        drain_writebacks(0)
        drain_gathers(1)
        fire_writebacks(g + 1, 1)
        drain_writebacks(1)

    return body


def kernel(input_ids, weight):
    B = input_ids.size
    D = weight.shape[1]
    idx = input_ids.reshape(B // _CHUNK, _CHUNK).astype(jnp.int32)
    out = _make_gather(B, D)(idx, weight)
    return out.reshape(*input_ids.shape, D)
